# baseline (speedup 1.0000x reference)
import functools

import jax
import jax.numpy as jnp
from jax import lax
from jax.experimental import pallas as pl
from jax.experimental.pallas import tpu as pltpu
from jax.experimental.pallas import tpu_sc as plsc

N_NODES = 10000
N_EDGES = 320000
D = 128
EDGE_DIM = 7
NUM_LAYERS = 3

NC = 2
NS = 16
NW = NC * NS
EPT = N_EDGES // NW
CH = 80
NCHUNK = EPT // CH
N_PAD = 10240
RPT = N_PAD // NS
NDB = D // 16

_MESH = plsc.VectorSubcoreMesh(core_axis_name="c", subcore_axis_name="s")


def _zero_fill(ref, nrows, ncols16):
    z = jnp.zeros((16,), jnp.float32)

    def body(i, _):
        for j in range(ncols16):
            ref[i, pl.ds(j * 16, 16)] = z
        return 0

    lax.fori_loop(0, nrows, body, 0)


CH2 = 2000


@functools.partial(
    pl.kernel,
    compiler_params=pltpu.CompilerParams(needs_layout_passes=False),
    out_type=jax.ShapeDtypeStruct((NC, NS, N_PAD // D, D), jnp.float32),
    mesh=_MESH,
    scratch_types=[
        pltpu.VMEM((CH2,), jnp.int32),
        pltpu.VMEM((N_PAD // D, D), jnp.float32),
    ],
)
def _sc_degree(row_hbm, out_hbm, ridx, table):
    cid = lax.axis_index("c")
    sid = lax.axis_index("s")
    wid = sid * NC + cid

    z = jnp.zeros((16,), jnp.float32)

    def zrow(i, _):
        for j in range(NDB):
            table[i, pl.ds(j * 16, 16)] = z
        return 0

    lax.fori_loop(0, N_PAD // D, zrow, 0)

    one = jnp.ones((16,), jnp.float32)

    def chunk(i, _):
        base = wid * EPT + i * CH2
        pltpu.sync_copy(row_hbm.at[pl.ds(base, CH2)], ridx)

        def step(k, _):
            iv = ridx[pl.ds(k * 16, 16)]
            r = iv // D
            c = iv - r * D
            plsc.addupdate_scatter(table, [r, c], one)
            return 0

        lax.fori_loop(0, CH2 // 16, step, 0)
        return 0

    lax.fori_loop(0, EPT // CH2, chunk, 0)
    pltpu.sync_copy(table, out_hbm.at[cid, sid])


@functools.partial(
    pl.kernel,
    compiler_params=pltpu.CompilerParams(needs_layout_passes=False),
    out_type=jax.ShapeDtypeStruct((N_EDGES,), jnp.float32),
    mesh=_MESH,
    scratch_types=[
        pltpu.VMEM((CH2,), jnp.int32),
        pltpu.VMEM((CH2,), jnp.float32),
        pltpu.VMEM((N_PAD,), jnp.float32),
    ],
)
def _sc_norm(row_hbm, dinv_hbm, out_hbm, ridx, nbuf, dinv):
    cid = lax.axis_index("c")
    sid = lax.axis_index("s")
    wid = sid * NC + cid
    pltpu.sync_copy(dinv_hbm, dinv)

    def chunk(i, _):
        base = wid * EPT + i * CH2
        pltpu.sync_copy(row_hbm.at[pl.ds(base, CH2)], ridx)

        def step(k, _):
            sl = pl.ds(k * 16, 16)
            nbuf[sl] = plsc.load_gather(dinv, [ridx[sl]])
            return 0

        lax.fori_loop(0, CH2 // 16, step, 0)
        pltpu.sync_copy(nbuf, out_hbm.at[pl.ds(base, CH2)])
        return 0

    lax.fori_loop(0, EPT // CH2, chunk, 0)


@functools.partial(
    pl.kernel,
    compiler_params=pltpu.CompilerParams(needs_layout_passes=False),
    out_type=jax.ShapeDtypeStruct((NC, N_PAD, D), jnp.float32),
    mesh=_MESH,
    scratch_types=[
        pltpu.VMEM((2, CH), jnp.int32),
        pltpu.VMEM((2, CH), jnp.int32),
        pltpu.VMEM((2, CH, D), jnp.float32),
        pltpu.VMEM((2, CH, D), jnp.float32),
        pltpu.VMEM_SHARED((N_PAD, D), jnp.float32),
        pltpu.SemaphoreType.DMA,
        pltpu.SemaphoreType.DMA,
        pltpu.SemaphoreType.DMA,
        pltpu.SemaphoreType.DMA,
    ],
)
def _sc_edge(hxs_hbm, ees_hbm, row_hbm, col_hbm, out_hbm,
             ridx, cidx, gbuf, ebuf, table, gsem0, gsem1, esem0, esem1):
    gsem = (gsem0, gsem1)
    esem = (esem0, esem1)
    cid = lax.axis_index("c")
    sid = lax.axis_index("s")
    wid = sid * NC + cid

    _zero_fill(gbuf.at[0], CH, NDB)
    for k in range(RPT // CH):
        pltpu.sync_copy(gbuf.at[0], table.at[pl.ds(sid * RPT + k * CH, CH)])
    plsc.subcore_barrier()

    def issue(j, b):
        base = wid * EPT + j * CH
        pltpu.sync_copy(row_hbm.at[pl.ds(base, CH)], ridx.at[b])
        pltpu.sync_copy(col_hbm.at[pl.ds(base, CH)], cidx.at[b])
        pltpu.async_copy(hxs_hbm.at[ridx.at[b]], gbuf.at[b], gsem[b])
        pltpu.async_copy(ees_hbm.at[pl.ds(base, CH)], ebuf.at[b], esem[b])

    def process(j, b):
        base = wid * EPT + j * CH
        pltpu.make_async_copy(hxs_hbm.at[ridx.at[b]], gbuf.at[b],
                              gsem[b]).wait()
        pltpu.make_async_copy(ees_hbm.at[pl.ds(base, CH)], ebuf.at[b],
                              esem[b]).wait()
        gb = gbuf.at[b]
        eb = ebuf.at[b]

        @plsc.parallel_loop(0, CH, unroll=2)
        def _(e):
            for jj in range(NDB):
                sl = pl.ds(jj * 16, 16)
                gb[e, sl] = jnp.maximum(gb[e, sl] + eb[e, sl], 0.0)

        pltpu.sync_copy(gbuf.at[b], table.at[cidx.at[b]], add=True)

        @pl.when(j + 2 < NCHUNK)
        def _():
            issue(j + 2, b)

    issue(0, 0)
    issue(1, 1)

    @pl.loop(0, NCHUNK - 1, step=2)
    def _(j):
        process(j, 0)
        process(j + 1, 1)

    process(NCHUNK - 1, 0)
    plsc.subcore_barrier()

    for k in range(RPT // CH):
        off = sid * RPT + k * CH
        pltpu.sync_copy(table.at[pl.ds(off, CH)], gbuf.at[0])
        pltpu.sync_copy(gbuf.at[0], out_hbm.at[cid, pl.ds(off, CH)])


def _deg_finish_body(degp_ref, deg_ref, dinv_ref):
    s = jnp.sum(degp_ref[...], axis=(0, 1)) + 1.0
    deg_ref[...] = s
    dinv_ref[...] = lax.rsqrt(s)


def _tc_deg_finish(degp):
    return pl.pallas_call(
        _deg_finish_body,
        out_shape=(
            jax.ShapeDtypeStruct((N_PAD // D, D), jnp.float32),
            jax.ShapeDtypeStruct((N_PAD // D, D), jnp.float32),
        ),
    )(degp)


def _linear_body(h_ref, w_ref, b_ref, dinv_ref, o_ref, os_ref):
    hx = (
        jnp.dot(h_ref[...], w_ref[...], preferred_element_type=jnp.float32)
        + b_ref[...]
    )
    o_ref[...] = hx
    os_ref[...] = hx * dinv_ref[...]


def _tc_linear(h, w, b, dinv_col):
    blk = 400
    return pl.pallas_call(
        _linear_body,
        grid=(N_NODES // blk,),
        in_specs=[
            pl.BlockSpec((blk, D), lambda i: (i, 0)),
            pl.BlockSpec((D, D), lambda i: (0, 0)),
            pl.BlockSpec((1, D), lambda i: (0, 0)),
            pl.BlockSpec((blk, 1), lambda i: (i, 0)),
        ],
        out_specs=(
            pl.BlockSpec((blk, D), lambda i: (i, 0)),
            pl.BlockSpec((blk, D), lambda i: (i, 0)),
        ),
        out_shape=(
            jax.ShapeDtypeStruct((N_NODES, D), jnp.float32),
            jax.ShapeDtypeStruct((N_NODES, D), jnp.float32),
        ),
    )(h, w, b.reshape(1, D), dinv_col)


def _ea_prep_body(ea_ref, nr_ref, o_ref):
    nr = nr_ref[...]
    blk = ea_ref.shape[0]
    m = (lax.broadcasted_iota(jnp.int32, (1, 8), 1) == 0).astype(jnp.float32)
    o_ref[...] = jnp.concatenate(
        [ea_ref[...] * nr, jnp.broadcast_to(m, (blk, 8)) * nr], axis=1)


def _tc_ea_prep(ea8, nr):
    blk = 4000
    return pl.pallas_call(
        _ea_prep_body,
        grid=(N_EDGES // blk,),
        in_specs=[
            pl.BlockSpec((blk, 8), lambda i: (i, 0)),
            pl.BlockSpec((blk, 1), lambda i: (i, 0)),
        ],
        out_specs=pl.BlockSpec((blk, 16), lambda i: (i, 0)),
        out_shape=jax.ShapeDtypeStruct((N_EDGES, 16), jnp.float32),
    )(ea8, nr)


def _edge_embed_body(ea_ref, w_ref, o_ref):
    o_ref[...] = jnp.dot(ea_ref[...], w_ref[...],
                         preferred_element_type=jnp.float32)


def _tc_edge_embed(ea16, w16):
    blk = 4000
    return pl.pallas_call(
        _edge_embed_body,
        grid=(N_EDGES // blk,),
        in_specs=[
            pl.BlockSpec((blk, 16), lambda i: (i, 0)),
            pl.BlockSpec((16, D), lambda i: (0, 0)),
        ],
        out_specs=pl.BlockSpec((blk, D), lambda i: (i, 0)),
        out_shape=jax.ShapeDtypeStruct((N_EDGES, D), jnp.float32),
    )(ea16, w16)


def _combine_body(part_ref, hx_ref, deg_ref, dinv_ref, root_ref, g_ref,
                  b_ref, o_ref, *, do_relu):
    hx = hx_ref[...]
    deg = deg_ref[:N_NODES, :1]
    dinv = dinv_ref[:N_NODES, :1]
    agg = (part_ref[0, :N_NODES] + part_ref[1, :N_NODES]) * dinv
    t = agg + jnp.maximum(hx + root_ref[...], 0.0) / deg
    m = jnp.mean(t, axis=0, keepdims=True)
    v = jnp.mean((t - m) * (t - m), axis=0, keepdims=True)
    out = (t - m) * lax.rsqrt(v + 1e-5) * g_ref[...] + b_ref[...]
    if do_relu:
        out = jnp.maximum(out, 0.0)
    o_ref[...] = out


def _tc_combine(part, hx, deg_col, dinv_col, root, g, b, do_relu):
    return pl.pallas_call(
        functools.partial(_combine_body, do_relu=do_relu),
        out_shape=jax.ShapeDtypeStruct((N_NODES, D), jnp.float32),
    )(part, hx, deg_col, dinv_col, root.reshape(1, D), g.reshape(1, D),
      b.reshape(1, D))


def kernel(x, edge_index, edge_attr, Wlin, blin, Wedge, bedge, root_emb,
           gamma, beta):
    row = edge_index[0]
    col = edge_index[1]
    ea8 = jnp.pad(edge_attr, ((0, 0), (0, 1)))
    zeros1 = jnp.zeros((NUM_LAYERS, 1, D), jnp.float32)
    zeros7 = jnp.zeros((NUM_LAYERS, 7, D), jnp.float32)
    W16 = jnp.concatenate([Wedge, zeros1, bedge[:, None, :], zeros7], axis=1)

    degp = _sc_degree(row)
    deg2d, dinv2d = _tc_deg_finish(degp)
    deg_col = deg2d.reshape(N_PAD, 1)
    dinv_col = dinv2d.reshape(N_PAD, 1)
    dinv_flat = dinv2d.reshape(N_PAD)
    norm_row = _sc_norm(row, dinv_flat).reshape(N_EDGES, 1)
    ea16 = _tc_ea_prep(ea8, norm_row)

    ees_all = [_tc_edge_embed(ea16, W16[l]) for l in range(NUM_LAYERS)]

    h = x
    for l in range(NUM_LAYERS):
        hx, hxs = _tc_linear(h, Wlin[l], blin[l], dinv_col)
        part = _sc_edge(hxs, ees_all[l], row, col)
        h = _tc_combine(part, hx, deg_col, dinv_col, root_emb[l], gamma[l],
                        beta[l], do_relu=(l < NUM_LAYERS - 1))
    return h


# speedup vs baseline: 1.0060x; 1.0060x over previous
import functools

import jax
import jax.numpy as jnp
from jax import lax
from jax.experimental import pallas as pl
from jax.experimental.pallas import tpu as pltpu
from jax.experimental.pallas import tpu_sc as plsc

N_NODES = 10000
N_EDGES = 320000
D = 128
EDGE_DIM = 7
NUM_LAYERS = 3

NC = 2
NS = 16
NW = NC * NS
EPT = N_EDGES // NW
CH = 80
NCHUNK = EPT // CH
N_PAD = 10240
RPT = N_PAD // NS
NDB = D // 16

_MESH = plsc.VectorSubcoreMesh(core_axis_name="c", subcore_axis_name="s")


def _zero_fill(ref, nrows, ncols16):
    z = jnp.zeros((16,), jnp.float32)

    def body(i, _):
        for j in range(ncols16):
            ref[i, pl.ds(j * 16, 16)] = z
        return 0

    lax.fori_loop(0, nrows, body, 0)


CH2 = 2000


@functools.partial(
    pl.kernel,
    compiler_params=pltpu.CompilerParams(needs_layout_passes=False),
    out_type=jax.ShapeDtypeStruct((NC, NS, N_PAD // D, D), jnp.float32),
    mesh=_MESH,
    scratch_types=[
        pltpu.VMEM((CH2,), jnp.int32),
        pltpu.VMEM((N_PAD // D, D), jnp.float32),
    ],
)
def _sc_degree(row_hbm, out_hbm, ridx, table):
    cid = lax.axis_index("c")
    sid = lax.axis_index("s")
    wid = sid * NC + cid

    z = jnp.zeros((16,), jnp.float32)

    def zrow(i, _):
        for j in range(NDB):
            table[i, pl.ds(j * 16, 16)] = z
        return 0

    lax.fori_loop(0, N_PAD // D, zrow, 0)

    one = jnp.ones((16,), jnp.float32)

    def chunk(i, _):
        base = wid * EPT + i * CH2
        pltpu.sync_copy(row_hbm.at[pl.ds(base, CH2)], ridx)

        def step(k, _):
            iv = ridx[pl.ds(k * 16, 16)]
            r = iv // D
            c = iv - r * D
            plsc.addupdate_scatter(table, [r, c], one)
            return 0

        lax.fori_loop(0, CH2 // 16, step, 0)
        return 0

    lax.fori_loop(0, EPT // CH2, chunk, 0)
    pltpu.sync_copy(table, out_hbm.at[cid, sid])


@functools.partial(
    pl.kernel,
    compiler_params=pltpu.CompilerParams(needs_layout_passes=False),
    out_type=jax.ShapeDtypeStruct((N_EDGES,), jnp.float32),
    mesh=_MESH,
    scratch_types=[
        pltpu.VMEM((CH2,), jnp.int32),
        pltpu.VMEM((CH2,), jnp.float32),
        pltpu.VMEM((N_PAD,), jnp.float32),
    ],
)
def _sc_norm(row_hbm, dinv_hbm, out_hbm, ridx, nbuf, dinv):
    cid = lax.axis_index("c")
    sid = lax.axis_index("s")
    wid = sid * NC + cid
    pltpu.sync_copy(dinv_hbm, dinv)

    def chunk(i, _):
        base = wid * EPT + i * CH2
        pltpu.sync_copy(row_hbm.at[pl.ds(base, CH2)], ridx)

        def step(k, _):
            sl = pl.ds(k * 16, 16)
            nbuf[sl] = plsc.load_gather(dinv, [ridx[sl]])
            return 0

        lax.fori_loop(0, CH2 // 16, step, 0)
        pltpu.sync_copy(nbuf, out_hbm.at[pl.ds(base, CH2)])
        return 0

    lax.fori_loop(0, EPT // CH2, chunk, 0)


@functools.partial(
    pl.kernel,
    compiler_params=pltpu.CompilerParams(needs_layout_passes=False),
    out_type=jax.ShapeDtypeStruct((NC, N_PAD, D), jnp.float32),
    mesh=_MESH,
    scratch_types=[
        pltpu.VMEM((2, CH), jnp.int32),
        pltpu.VMEM((2, CH), jnp.int32),
        pltpu.VMEM((2, CH, D), jnp.float32),
        pltpu.VMEM((2, CH, D), jnp.float32),
        pltpu.VMEM_SHARED((N_PAD, D), jnp.float32),
        pltpu.SemaphoreType.DMA,
        pltpu.SemaphoreType.DMA,
        pltpu.SemaphoreType.DMA,
        pltpu.SemaphoreType.DMA,
    ],
)
def _sc_edge(hxs_hbm, ees_hbm, row_hbm, col_hbm, out_hbm,
             ridx, cidx, gbuf, ebuf, table, gsem0, gsem1, esem0, esem1):
    gsem = (gsem0, gsem1)
    esem = (esem0, esem1)
    cid = lax.axis_index("c")
    sid = lax.axis_index("s")
    wid = sid * NC + cid

    _zero_fill(gbuf.at[0], CH, NDB)
    for k in range(RPT // CH):
        pltpu.sync_copy(gbuf.at[0], table.at[pl.ds(sid * RPT + k * CH, CH)])
    plsc.subcore_barrier()

    def issue(j, b):
        base = wid * EPT + j * CH
        pltpu.sync_copy(row_hbm.at[pl.ds(base, CH)], ridx.at[b])
        pltpu.sync_copy(col_hbm.at[pl.ds(base, CH)], cidx.at[b])
        pltpu.async_copy(hxs_hbm.at[ridx.at[b]], gbuf.at[b], gsem[b])
        pltpu.async_copy(ees_hbm.at[pl.ds(base, CH)], ebuf.at[b], esem[b])

    def process(j, b):
        base = wid * EPT + j * CH
        pltpu.make_async_copy(hxs_hbm.at[ridx.at[b]], gbuf.at[b],
                              gsem[b]).wait()
        pltpu.make_async_copy(ees_hbm.at[pl.ds(base, CH)], ebuf.at[b],
                              esem[b]).wait()
        gb = gbuf.at[b]
        eb = ebuf.at[b]

        @plsc.parallel_loop(0, CH, unroll=2)
        def _(e):
            for jj in range(NDB):
                sl = pl.ds(jj * 16, 16)
                gb[e, sl] = jnp.maximum(gb[e, sl] + eb[e, sl], 0.0)

        pltpu.sync_copy(gbuf.at[b], table.at[cidx.at[b]], add=True)

        @pl.when(j + 2 < NCHUNK)
        def _():
            issue(j + 2, b)

    issue(0, 0)
    issue(1, 1)

    @pl.loop(0, NCHUNK - 1, step=2)
    def _(j):
        process(j, 0)
        process(j + 1, 1)

    process(NCHUNK - 1, 0)
    plsc.subcore_barrier()

    for k in range(RPT // CH):
        off = sid * RPT + k * CH
        pltpu.sync_copy(table.at[pl.ds(off, CH)], gbuf.at[0])
        pltpu.sync_copy(gbuf.at[0], out_hbm.at[cid, pl.ds(off, CH)])


def _deg_finish_body(degp_ref, deg_ref, dinv_ref):
    s = jnp.sum(degp_ref[...], axis=(0, 1)) + 1.0
    deg_ref[...] = s
    dinv_ref[...] = lax.rsqrt(s)


def _tc_deg_finish(degp):
    return pl.pallas_call(
        _deg_finish_body,
        out_shape=(
            jax.ShapeDtypeStruct((N_PAD // D, D), jnp.float32),
            jax.ShapeDtypeStruct((N_PAD // D, D), jnp.float32),
        ),
    )(degp)


def _linear_body(h_ref, w_ref, b_ref, dinv_ref, o_ref, os_ref):
    hx = (
        jnp.dot(h_ref[...], w_ref[...], preferred_element_type=jnp.float32)
        + b_ref[...]
    )
    o_ref[...] = hx
    os_ref[...] = hx * dinv_ref[...]


def _tc_linear(h, w, b, dinv_col):
    blk = 1000
    return pl.pallas_call(
        _linear_body,
        grid=(N_NODES // blk,),
        in_specs=[
            pl.BlockSpec((blk, D), lambda i: (i, 0)),
            pl.BlockSpec((D, D), lambda i: (0, 0)),
            pl.BlockSpec((1, D), lambda i: (0, 0)),
            pl.BlockSpec((blk, 1), lambda i: (i, 0)),
        ],
        out_specs=(
            pl.BlockSpec((blk, D), lambda i: (i, 0)),
            pl.BlockSpec((blk, D), lambda i: (i, 0)),
        ),
        out_shape=(
            jax.ShapeDtypeStruct((N_NODES, D), jnp.float32),
            jax.ShapeDtypeStruct((N_NODES, D), jnp.float32),
        ),
    )(h, w, b.reshape(1, D), dinv_col)


def _ea_prep_body(ea_ref, nr_ref, o_ref):
    nr = nr_ref[...]
    blk = ea_ref.shape[0]
    m = (lax.broadcasted_iota(jnp.int32, (1, 8), 1) == 0).astype(jnp.float32)
    o_ref[...] = jnp.concatenate(
        [ea_ref[...] * nr, jnp.broadcast_to(m, (blk, 8)) * nr], axis=1)


def _tc_ea_prep(ea8, nr):
    blk = 4000
    return pl.pallas_call(
        _ea_prep_body,
        grid=(N_EDGES // blk,),
        in_specs=[
            pl.BlockSpec((blk, 8), lambda i: (i, 0)),
            pl.BlockSpec((blk, 1), lambda i: (i, 0)),
        ],
        out_specs=pl.BlockSpec((blk, 16), lambda i: (i, 0)),
        out_shape=jax.ShapeDtypeStruct((N_EDGES, 16), jnp.float32),
    )(ea8, nr)


def _edge_embed_body(ea_ref, w_ref, o_ref):
    o_ref[...] = jnp.dot(ea_ref[...], w_ref[...],
                         preferred_element_type=jnp.float32)


def _tc_edge_embed(ea16, w16):
    blk = 10000
    return pl.pallas_call(
        _edge_embed_body,
        grid=(N_EDGES // blk,),
        in_specs=[
            pl.BlockSpec((blk, 16), lambda i: (i, 0)),
            pl.BlockSpec((16, D), lambda i: (0, 0)),
        ],
        out_specs=pl.BlockSpec((blk, D), lambda i: (i, 0)),
        out_shape=jax.ShapeDtypeStruct((N_EDGES, D), jnp.float32),
    )(ea16, w16)


def _combine_body(part_ref, hx_ref, deg_ref, dinv_ref, root_ref, g_ref,
                  b_ref, o_ref, *, do_relu):
    hx = hx_ref[...]
    deg = deg_ref[:N_NODES, :1]
    dinv = dinv_ref[:N_NODES, :1]
    agg = (part_ref[0, :N_NODES] + part_ref[1, :N_NODES]) * dinv
    t = agg + jnp.maximum(hx + root_ref[...], 0.0) / deg
    m = jnp.mean(t, axis=0, keepdims=True)
    v = jnp.mean((t - m) * (t - m), axis=0, keepdims=True)
    out = (t - m) * lax.rsqrt(v + 1e-5) * g_ref[...] + b_ref[...]
    if do_relu:
        out = jnp.maximum(out, 0.0)
    o_ref[...] = out


def _tc_combine(part, hx, deg_col, dinv_col, root, g, b, do_relu):
    return pl.pallas_call(
        functools.partial(_combine_body, do_relu=do_relu),
        out_shape=jax.ShapeDtypeStruct((N_NODES, D), jnp.float32),
    )(part, hx, deg_col, dinv_col, root.reshape(1, D), g.reshape(1, D),
      b.reshape(1, D))


def kernel(x, edge_index, edge_attr, Wlin, blin, Wedge, bedge, root_emb,
           gamma, beta):
    row = edge_index[0]
    col = edge_index[1]
    ea8 = jnp.pad(edge_attr, ((0, 0), (0, 1)))
    zeros1 = jnp.zeros((NUM_LAYERS, 1, D), jnp.float32)
    zeros7 = jnp.zeros((NUM_LAYERS, 7, D), jnp.float32)
    W16 = jnp.concatenate([Wedge, zeros1, bedge[:, None, :], zeros7], axis=1)

    degp = _sc_degree(row)
    deg2d, dinv2d = _tc_deg_finish(degp)
    deg_col = deg2d.reshape(N_PAD, 1)
    dinv_col = dinv2d.reshape(N_PAD, 1)
    dinv_flat = dinv2d.reshape(N_PAD)
    norm_row = _sc_norm(row, dinv_flat).reshape(N_EDGES, 1)
    ea16 = _tc_ea_prep(ea8, norm_row)

    ees_all = [_tc_edge_embed(ea16, W16[l]) for l in range(NUM_LAYERS)]

    h = x
    for l in range(NUM_LAYERS):
        hx, hxs = _tc_linear(h, Wlin[l], blin[l], dinv_col)
        part = _sc_edge(hxs, ees_all[l], row, col)
        h = _tc_combine(part, hx, deg_col, dinv_col, root_emb[l], gamma[l],
                        beta[l], do_relu=(l < NUM_LAYERS - 1))
    return h


# speedup vs baseline: 1.0105x; 1.0045x over previous
import functools

import jax
import jax.numpy as jnp
from jax import lax
from jax.experimental import pallas as pl
from jax.experimental.pallas import tpu as pltpu
from jax.experimental.pallas import tpu_sc as plsc

N_NODES = 10000
N_EDGES = 320000
D = 128
EDGE_DIM = 7
NUM_LAYERS = 3

NC = 2
NS = 16
NW = NC * NS
EPT = N_EDGES // NW
CH = 80
NCHUNK = EPT // CH
N_PAD = 10240
RPT = N_PAD // NS
NDB = D // 16

_MESH = plsc.VectorSubcoreMesh(core_axis_name="c", subcore_axis_name="s")


def _zero_fill(ref, nrows, ncols16):
    z = jnp.zeros((16,), jnp.float32)

    def body(i, _):
        for j in range(ncols16):
            ref[i, pl.ds(j * 16, 16)] = z
        return 0

    lax.fori_loop(0, nrows, body, 0)


CH2 = 2000


@functools.partial(
    pl.kernel,
    compiler_params=pltpu.CompilerParams(needs_layout_passes=False),
    out_type=jax.ShapeDtypeStruct((NC, NS, N_PAD // D, D), jnp.float32),
    mesh=_MESH,
    scratch_types=[
        pltpu.VMEM((CH2,), jnp.int32),
        pltpu.VMEM((N_PAD // D, D), jnp.float32),
    ],
)
def _sc_degree(row_hbm, out_hbm, ridx, table):
    cid = lax.axis_index("c")
    sid = lax.axis_index("s")
    wid = sid * NC + cid

    z = jnp.zeros((16,), jnp.float32)

    def zrow(i, _):
        for j in range(NDB):
            table[i, pl.ds(j * 16, 16)] = z
        return 0

    lax.fori_loop(0, N_PAD // D, zrow, 0)

    one = jnp.ones((16,), jnp.float32)

    def chunk(i, _):
        base = wid * EPT + i * CH2
        pltpu.sync_copy(row_hbm.at[pl.ds(base, CH2)], ridx)

        def step(k, _):
            iv = ridx[pl.ds(k * 16, 16)]
            r = iv // D
            c = iv - r * D
            plsc.addupdate_scatter(table, [r, c], one)
            return 0

        lax.fori_loop(0, CH2 // 16, step, 0)
        return 0

    lax.fori_loop(0, EPT // CH2, chunk, 0)
    pltpu.sync_copy(table, out_hbm.at[cid, sid])


@functools.partial(
    pl.kernel,
    compiler_params=pltpu.CompilerParams(needs_layout_passes=False),
    out_type=jax.ShapeDtypeStruct((N_EDGES,), jnp.float32),
    mesh=_MESH,
    scratch_types=[
        pltpu.VMEM((CH2,), jnp.int32),
        pltpu.VMEM((CH2,), jnp.float32),
        pltpu.VMEM((N_PAD,), jnp.float32),
    ],
)
def _sc_norm(row_hbm, dinv_hbm, out_hbm, ridx, nbuf, dinv):
    cid = lax.axis_index("c")
    sid = lax.axis_index("s")
    wid = sid * NC + cid
    pltpu.sync_copy(dinv_hbm, dinv)

    def chunk(i, _):
        base = wid * EPT + i * CH2
        pltpu.sync_copy(row_hbm.at[pl.ds(base, CH2)], ridx)

        def step(k, _):
            sl = pl.ds(k * 16, 16)
            nbuf[sl] = plsc.load_gather(dinv, [ridx[sl]])
            return 0

        lax.fori_loop(0, CH2 // 16, step, 0)
        pltpu.sync_copy(nbuf, out_hbm.at[pl.ds(base, CH2)])
        return 0

    lax.fori_loop(0, EPT // CH2, chunk, 0)


@functools.partial(
    pl.kernel,
    compiler_params=pltpu.CompilerParams(needs_layout_passes=False),
    out_type=jax.ShapeDtypeStruct((NC, N_PAD, D), jnp.float32),
    mesh=_MESH,
    scratch_types=[
        pltpu.VMEM((2, CH), jnp.int32),
        pltpu.VMEM((2, CH), jnp.int32),
        pltpu.VMEM((2, CH, D), jnp.float32),
        pltpu.VMEM((2, CH, D), jnp.float32),
        pltpu.VMEM_SHARED((N_PAD, D), jnp.float32),
        pltpu.SemaphoreType.DMA,
        pltpu.SemaphoreType.DMA,
        pltpu.SemaphoreType.DMA,
        pltpu.SemaphoreType.DMA,
    ],
)
def _sc_edge(hxs_hbm, ees_hbm, row_hbm, col_hbm, out_hbm,
             ridx, cidx, gbuf, ebuf, table, gsem0, gsem1, esem0, esem1):
    gsem = (gsem0, gsem1)
    esem = (esem0, esem1)
    cid = lax.axis_index("c")
    sid = lax.axis_index("s")
    wid = sid * NC + cid

    _zero_fill(gbuf.at[0], CH, NDB)
    for k in range(RPT // CH):
        pltpu.sync_copy(gbuf.at[0], table.at[pl.ds(sid * RPT + k * CH, CH)])
    plsc.subcore_barrier()

    def issue(j, b):
        base = wid * EPT + j * CH
        pltpu.sync_copy(row_hbm.at[pl.ds(base, CH)], ridx.at[b])
        pltpu.sync_copy(col_hbm.at[pl.ds(base, CH)], cidx.at[b])
        pltpu.async_copy(hxs_hbm.at[ridx.at[b]], gbuf.at[b], gsem[b])
        pltpu.async_copy(ees_hbm.at[pl.ds(base, CH)], ebuf.at[b], esem[b])

    def process(j, b):
        base = wid * EPT + j * CH
        pltpu.make_async_copy(hxs_hbm.at[ridx.at[b]], gbuf.at[b],
                              gsem[b]).wait()
        pltpu.make_async_copy(ees_hbm.at[pl.ds(base, CH)], ebuf.at[b],
                              esem[b]).wait()
        gb = gbuf.at[b]
        eb = ebuf.at[b]

        @plsc.parallel_loop(0, CH, unroll=2)
        def _(e):
            for jj in range(NDB):
                sl = pl.ds(jj * 16, 16)
                gb[e, sl] = jnp.maximum(gb[e, sl] + eb[e, sl], 0.0)

        pltpu.sync_copy(gbuf.at[b], table.at[cidx.at[b]], add=True)

        @pl.when(j + 2 < NCHUNK)
        def _():
            issue(j + 2, b)

    issue(0, 0)
    issue(1, 1)

    @pl.loop(0, NCHUNK - 1, step=2)
    def _(j):
        process(j, 0)
        process(j + 1, 1)

    process(NCHUNK - 1, 0)
    plsc.subcore_barrier()

    for k in range(RPT // CH):
        off = sid * RPT + k * CH
        pltpu.sync_copy(table.at[pl.ds(off, CH)], gbuf.at[0])
        pltpu.sync_copy(gbuf.at[0], out_hbm.at[cid, pl.ds(off, CH)])


def _deg_finish_body(degp_ref, deg_ref, dinv_ref):
    s = jnp.sum(degp_ref[...], axis=(0, 1)) + 1.0
    deg_ref[...] = s
    dinv_ref[...] = lax.rsqrt(s)


def _tc_deg_finish(degp):
    return pl.pallas_call(
        _deg_finish_body,
        out_shape=(
            jax.ShapeDtypeStruct((N_PAD // D, D), jnp.float32),
            jax.ShapeDtypeStruct((N_PAD // D, D), jnp.float32),
        ),
    )(degp)


def _linear_body(h_ref, w_ref, b_ref, dinv_ref, o_ref, os_ref):
    hx = (
        jnp.dot(h_ref[...], w_ref[...], preferred_element_type=jnp.float32)
        + b_ref[...]
    )
    o_ref[...] = hx
    os_ref[...] = hx * dinv_ref[...]


def _tc_linear(h, w, b, dinv_col):
    blk = 2000
    return pl.pallas_call(
        _linear_body,
        grid=(N_NODES // blk,),
        in_specs=[
            pl.BlockSpec((blk, D), lambda i: (i, 0)),
            pl.BlockSpec((D, D), lambda i: (0, 0)),
            pl.BlockSpec((1, D), lambda i: (0, 0)),
            pl.BlockSpec((blk, 1), lambda i: (i, 0)),
        ],
        out_specs=(
            pl.BlockSpec((blk, D), lambda i: (i, 0)),
            pl.BlockSpec((blk, D), lambda i: (i, 0)),
        ),
        out_shape=(
            jax.ShapeDtypeStruct((N_NODES, D), jnp.float32),
            jax.ShapeDtypeStruct((N_NODES, D), jnp.float32),
        ),
    )(h, w, b.reshape(1, D), dinv_col)


def _ea_prep_body(ea_ref, nr_ref, o_ref):
    nr = nr_ref[...]
    blk = ea_ref.shape[0]
    m = (lax.broadcasted_iota(jnp.int32, (1, 8), 1) == 0).astype(jnp.float32)
    o_ref[...] = jnp.concatenate(
        [ea_ref[...] * nr, jnp.broadcast_to(m, (blk, 8)) * nr], axis=1)


def _tc_ea_prep(ea8, nr):
    blk = 4000
    return pl.pallas_call(
        _ea_prep_body,
        grid=(N_EDGES // blk,),
        in_specs=[
            pl.BlockSpec((blk, 8), lambda i: (i, 0)),
            pl.BlockSpec((blk, 1), lambda i: (i, 0)),
        ],
        out_specs=pl.BlockSpec((blk, 16), lambda i: (i, 0)),
        out_shape=jax.ShapeDtypeStruct((N_EDGES, 16), jnp.float32),
    )(ea8, nr)


def _edge_embed_body(ea_ref, w_ref, o_ref):
    o_ref[...] = jnp.dot(ea_ref[...], w_ref[...],
                         preferred_element_type=jnp.float32)


def _tc_edge_embed(ea16, w16):
    blk = 16000
    return pl.pallas_call(
        _edge_embed_body,
        grid=(N_EDGES // blk,),
        in_specs=[
            pl.BlockSpec((blk, 16), lambda i: (i, 0)),
            pl.BlockSpec((16, D), lambda i: (0, 0)),
        ],
        out_specs=pl.BlockSpec((blk, D), lambda i: (i, 0)),
        out_shape=jax.ShapeDtypeStruct((N_EDGES, D), jnp.float32),
    )(ea16, w16)


def _combine_body(part_ref, hx_ref, deg_ref, dinv_ref, root_ref, g_ref,
                  b_ref, o_ref, *, do_relu):
    hx = hx_ref[...]
    deg = deg_ref[:N_NODES, :1]
    dinv = dinv_ref[:N_NODES, :1]
    agg = (part_ref[0, :N_NODES] + part_ref[1, :N_NODES]) * dinv
    t = agg + jnp.maximum(hx + root_ref[...], 0.0) / deg
    m = jnp.mean(t, axis=0, keepdims=True)
    v = jnp.mean((t - m) * (t - m), axis=0, keepdims=True)
    out = (t - m) * lax.rsqrt(v + 1e-5) * g_ref[...] + b_ref[...]
    if do_relu:
        out = jnp.maximum(out, 0.0)
    o_ref[...] = out


def _tc_combine(part, hx, deg_col, dinv_col, root, g, b, do_relu):
    return pl.pallas_call(
        functools.partial(_combine_body, do_relu=do_relu),
        out_shape=jax.ShapeDtypeStruct((N_NODES, D), jnp.float32),
    )(part, hx, deg_col, dinv_col, root.reshape(1, D), g.reshape(1, D),
      b.reshape(1, D))


def kernel(x, edge_index, edge_attr, Wlin, blin, Wedge, bedge, root_emb,
           gamma, beta):
    row = edge_index[0]
    col = edge_index[1]
    ea8 = jnp.pad(edge_attr, ((0, 0), (0, 1)))
    zeros1 = jnp.zeros((NUM_LAYERS, 1, D), jnp.float32)
    zeros7 = jnp.zeros((NUM_LAYERS, 7, D), jnp.float32)
    W16 = jnp.concatenate([Wedge, zeros1, bedge[:, None, :], zeros7], axis=1)

    degp = _sc_degree(row)
    deg2d, dinv2d = _tc_deg_finish(degp)
    deg_col = deg2d.reshape(N_PAD, 1)
    dinv_col = dinv2d.reshape(N_PAD, 1)
    dinv_flat = dinv2d.reshape(N_PAD)
    norm_row = _sc_norm(row, dinv_flat).reshape(N_EDGES, 1)
    ea16 = _tc_ea_prep(ea8, norm_row)

    ees_all = [_tc_edge_embed(ea16, W16[l]) for l in range(NUM_LAYERS)]

    h = x
    for l in range(NUM_LAYERS):
        hx, hxs = _tc_linear(h, Wlin[l], blin[l], dinv_col)
        part = _sc_edge(hxs, ees_all[l], row, col)
        h = _tc_combine(part, hx, deg_col, dinv_col, root_emb[l], gamma[l],
                        beta[l], do_relu=(l < NUM_LAYERS - 1))
    return h


# speedup vs baseline: 1.0161x; 1.0055x over previous
import functools

import jax
import jax.numpy as jnp
from jax import lax
from jax.experimental import pallas as pl
from jax.experimental.pallas import tpu as pltpu
from jax.experimental.pallas import tpu_sc as plsc

N_NODES = 10000
N_EDGES = 320000
D = 128
EDGE_DIM = 7
NUM_LAYERS = 3

NC = 2
NS = 16
NW = NC * NS
EPT = N_EDGES // NW
CH = 80
NCHUNK = EPT // CH
N_PAD = 10240
RPT = N_PAD // NS
NDB = D // 16

_MESH = plsc.VectorSubcoreMesh(core_axis_name="c", subcore_axis_name="s")


def _zero_fill(ref, nrows, ncols16):
    z = jnp.zeros((16,), jnp.float32)

    def body(i, _):
        for j in range(ncols16):
            ref[i, pl.ds(j * 16, 16)] = z
        return 0

    lax.fori_loop(0, nrows, body, 0)


CH2 = 2000


@functools.partial(
    pl.kernel,
    compiler_params=pltpu.CompilerParams(needs_layout_passes=False),
    out_type=jax.ShapeDtypeStruct((NC, NS, N_PAD // D, D), jnp.float32),
    mesh=_MESH,
    scratch_types=[
        pltpu.VMEM((CH2,), jnp.int32),
        pltpu.VMEM((N_PAD // D, D), jnp.float32),
    ],
)
def _sc_degree(row_hbm, out_hbm, ridx, table):
    cid = lax.axis_index("c")
    sid = lax.axis_index("s")
    wid = sid * NC + cid

    z = jnp.zeros((16,), jnp.float32)

    def zrow(i, _):
        for j in range(NDB):
            table[i, pl.ds(j * 16, 16)] = z
        return 0

    lax.fori_loop(0, N_PAD // D, zrow, 0)

    one = jnp.ones((16,), jnp.float32)

    def chunk(i, _):
        base = wid * EPT + i * CH2
        pltpu.sync_copy(row_hbm.at[pl.ds(base, CH2)], ridx)

        def step(k, _):
            iv = ridx[pl.ds(k * 16, 16)]
            r = iv // D
            c = iv - r * D
            plsc.addupdate_scatter(table, [r, c], one)
            return 0

        lax.fori_loop(0, CH2 // 16, step, 0)
        return 0

    lax.fori_loop(0, EPT // CH2, chunk, 0)
    pltpu.sync_copy(table, out_hbm.at[cid, sid])


@functools.partial(
    pl.kernel,
    compiler_params=pltpu.CompilerParams(needs_layout_passes=False),
    out_type=jax.ShapeDtypeStruct((N_EDGES,), jnp.float32),
    mesh=_MESH,
    scratch_types=[
        pltpu.VMEM((CH2,), jnp.int32),
        pltpu.VMEM((CH2,), jnp.float32),
        pltpu.VMEM((N_PAD,), jnp.float32),
    ],
)
def _sc_norm(row_hbm, dinv_hbm, out_hbm, ridx, nbuf, dinv):
    cid = lax.axis_index("c")
    sid = lax.axis_index("s")
    wid = sid * NC + cid
    pltpu.sync_copy(dinv_hbm, dinv)

    def chunk(i, _):
        base = wid * EPT + i * CH2
        pltpu.sync_copy(row_hbm.at[pl.ds(base, CH2)], ridx)

        def step(k, _):
            sl = pl.ds(k * 16, 16)
            nbuf[sl] = plsc.load_gather(dinv, [ridx[sl]])
            return 0

        lax.fori_loop(0, CH2 // 16, step, 0)
        pltpu.sync_copy(nbuf, out_hbm.at[pl.ds(base, CH2)])
        return 0

    lax.fori_loop(0, EPT // CH2, chunk, 0)


@functools.partial(
    pl.kernel,
    compiler_params=pltpu.CompilerParams(needs_layout_passes=False),
    out_type=jax.ShapeDtypeStruct((NC, N_PAD, D), jnp.float32),
    mesh=_MESH,
    scratch_types=[
        pltpu.VMEM((2, CH), jnp.int32),
        pltpu.VMEM((2, CH), jnp.int32),
        pltpu.VMEM((2, CH, D), jnp.float32),
        pltpu.VMEM((2, CH, D), jnp.float32),
        pltpu.VMEM_SHARED((N_PAD, D), jnp.float32),
        pltpu.SemaphoreType.DMA,
        pltpu.SemaphoreType.DMA,
        pltpu.SemaphoreType.DMA,
        pltpu.SemaphoreType.DMA,
    ],
)
def _sc_edge(hxs_hbm, ees_hbm, row_hbm, col_hbm, out_hbm,
             ridx, cidx, gbuf, ebuf, table, gsem0, gsem1, esem0, esem1):
    gsem = (gsem0, gsem1)
    esem = (esem0, esem1)
    cid = lax.axis_index("c")
    sid = lax.axis_index("s")
    wid = sid * NC + cid

    _zero_fill(gbuf.at[0], CH, NDB)
    for k in range(RPT // CH):
        pltpu.sync_copy(gbuf.at[0], table.at[pl.ds(sid * RPT + k * CH, CH)])
    plsc.subcore_barrier()

    def issue(j, b):
        base = wid * EPT + j * CH
        pltpu.sync_copy(row_hbm.at[pl.ds(base, CH)], ridx.at[b])
        pltpu.sync_copy(col_hbm.at[pl.ds(base, CH)], cidx.at[b])
        pltpu.async_copy(hxs_hbm.at[ridx.at[b]], gbuf.at[b], gsem[b])
        pltpu.async_copy(ees_hbm.at[pl.ds(base, CH)], ebuf.at[b], esem[b])

    def process(j, b):
        base = wid * EPT + j * CH
        pltpu.make_async_copy(hxs_hbm.at[ridx.at[b]], gbuf.at[b],
                              gsem[b]).wait()
        pltpu.make_async_copy(ees_hbm.at[pl.ds(base, CH)], ebuf.at[b],
                              esem[b]).wait()
        gb = gbuf.at[b]
        eb = ebuf.at[b]

        @plsc.parallel_loop(0, CH, unroll=2)
        def _(e):
            for jj in range(NDB):
                sl = pl.ds(jj * 16, 16)
                gb[e, sl] = jnp.maximum(gb[e, sl] + eb[e, sl], 0.0)

        pltpu.sync_copy(gbuf.at[b], table.at[cidx.at[b]], add=True)

        @pl.when(j + 2 < NCHUNK)
        def _():
            issue(j + 2, b)

    issue(0, 0)
    issue(1, 1)

    @pl.loop(0, NCHUNK - 1, step=2)
    def _(j):
        process(j, 0)
        process(j + 1, 1)

    process(NCHUNK - 1, 0)
    plsc.subcore_barrier()

    for k in range(RPT // CH):
        off = sid * RPT + k * CH
        pltpu.sync_copy(table.at[pl.ds(off, CH)], gbuf.at[0])
        pltpu.sync_copy(gbuf.at[0], out_hbm.at[cid, pl.ds(off, CH)])


def _deg_finish_body(degp_ref, deg_ref, dinv_ref):
    s = jnp.sum(degp_ref[...], axis=(0, 1)) + 1.0
    deg_ref[...] = s
    dinv_ref[...] = lax.rsqrt(s)


def _tc_deg_finish(degp):
    return pl.pallas_call(
        _deg_finish_body,
        out_shape=(
            jax.ShapeDtypeStruct((N_PAD // D, D), jnp.float32),
            jax.ShapeDtypeStruct((N_PAD // D, D), jnp.float32),
        ),
    )(degp)


def _linear_body(h_ref, w_ref, b_ref, dinv_ref, o_ref, os_ref):
    hx = (
        jnp.dot(h_ref[...], w_ref[...], preferred_element_type=jnp.float32)
        + b_ref[...]
    )
    o_ref[...] = hx
    os_ref[...] = hx * dinv_ref[...]


def _tc_linear(h, w, b, dinv_col):
    blk = 5000
    return pl.pallas_call(
        _linear_body,
        grid=(N_NODES // blk,),
        in_specs=[
            pl.BlockSpec((blk, D), lambda i: (i, 0)),
            pl.BlockSpec((D, D), lambda i: (0, 0)),
            pl.BlockSpec((1, D), lambda i: (0, 0)),
            pl.BlockSpec((blk, 1), lambda i: (i, 0)),
        ],
        out_specs=(
            pl.BlockSpec((blk, D), lambda i: (i, 0)),
            pl.BlockSpec((blk, D), lambda i: (i, 0)),
        ),
        out_shape=(
            jax.ShapeDtypeStruct((N_NODES, D), jnp.float32),
            jax.ShapeDtypeStruct((N_NODES, D), jnp.float32),
        ),
    )(h, w, b.reshape(1, D), dinv_col)


def _ea_prep_body(ea_ref, nr_ref, o_ref):
    nr = nr_ref[...]
    blk = ea_ref.shape[0]
    m = (lax.broadcasted_iota(jnp.int32, (1, 8), 1) == 0).astype(jnp.float32)
    o_ref[...] = jnp.concatenate(
        [ea_ref[...] * nr, jnp.broadcast_to(m, (blk, 8)) * nr], axis=1)


def _tc_ea_prep(ea8, nr):
    blk = 4000
    return pl.pallas_call(
        _ea_prep_body,
        grid=(N_EDGES // blk,),
        in_specs=[
            pl.BlockSpec((blk, 8), lambda i: (i, 0)),
            pl.BlockSpec((blk, 1), lambda i: (i, 0)),
        ],
        out_specs=pl.BlockSpec((blk, 16), lambda i: (i, 0)),
        out_shape=jax.ShapeDtypeStruct((N_EDGES, 16), jnp.float32),
    )(ea8, nr)


def _edge_embed_body(ea_ref, w_ref, o_ref):
    o_ref[...] = jnp.dot(ea_ref[...], w_ref[...],
                         preferred_element_type=jnp.float32)


def _tc_edge_embed(ea16, w16):
    blk = 20000
    return pl.pallas_call(
        _edge_embed_body,
        grid=(N_EDGES // blk,),
        in_specs=[
            pl.BlockSpec((blk, 16), lambda i: (i, 0)),
            pl.BlockSpec((16, D), lambda i: (0, 0)),
        ],
        out_specs=pl.BlockSpec((blk, D), lambda i: (i, 0)),
        out_shape=jax.ShapeDtypeStruct((N_EDGES, D), jnp.float32),
    )(ea16, w16)


def _combine_body(part_ref, hx_ref, deg_ref, dinv_ref, root_ref, g_ref,
                  b_ref, o_ref, *, do_relu):
    hx = hx_ref[...]
    deg = deg_ref[:N_NODES, :1]
    dinv = dinv_ref[:N_NODES, :1]
    agg = (part_ref[0, :N_NODES] + part_ref[1, :N_NODES]) * dinv
    t = agg + jnp.maximum(hx + root_ref[...], 0.0) / deg
    m = jnp.mean(t, axis=0, keepdims=True)
    v = jnp.mean((t - m) * (t - m), axis=0, keepdims=True)
    out = (t - m) * lax.rsqrt(v + 1e-5) * g_ref[...] + b_ref[...]
    if do_relu:
        out = jnp.maximum(out, 0.0)
    o_ref[...] = out


def _tc_combine(part, hx, deg_col, dinv_col, root, g, b, do_relu):
    return pl.pallas_call(
        functools.partial(_combine_body, do_relu=do_relu),
        out_shape=jax.ShapeDtypeStruct((N_NODES, D), jnp.float32),
    )(part, hx, deg_col, dinv_col, root.reshape(1, D), g.reshape(1, D),
      b.reshape(1, D))


def kernel(x, edge_index, edge_attr, Wlin, blin, Wedge, bedge, root_emb,
           gamma, beta):
    row = edge_index[0]
    col = edge_index[1]
    ea8 = jnp.pad(edge_attr, ((0, 0), (0, 1)))
    zeros1 = jnp.zeros((NUM_LAYERS, 1, D), jnp.float32)
    zeros7 = jnp.zeros((NUM_LAYERS, 7, D), jnp.float32)
    W16 = jnp.concatenate([Wedge, zeros1, bedge[:, None, :], zeros7], axis=1)

    degp = _sc_degree(row)
    deg2d, dinv2d = _tc_deg_finish(degp)
    deg_col = deg2d.reshape(N_PAD, 1)
    dinv_col = dinv2d.reshape(N_PAD, 1)
    dinv_flat = dinv2d.reshape(N_PAD)
    norm_row = _sc_norm(row, dinv_flat).reshape(N_EDGES, 1)
    ea16 = _tc_ea_prep(ea8, norm_row)

    ees_all = [_tc_edge_embed(ea16, W16[l]) for l in range(NUM_LAYERS)]

    h = x
    for l in range(NUM_LAYERS):
        hx, hxs = _tc_linear(h, Wlin[l], blin[l], dinv_col)
        part = _sc_edge(hxs, ees_all[l], row, col)
        h = _tc_combine(part, hx, deg_col, dinv_col, root_emb[l], gamma[l],
                        beta[l], do_relu=(l < NUM_LAYERS - 1))
    return h
